# baseline (speedup 1.0000x reference)
import functools

import jax
import jax.numpy as jnp
from jax import lax
from jax.experimental import pallas as pl
from jax.experimental.pallas import tpu as pltpu
from jax.experimental.pallas import tpu_sc as plsc

VOCAB = 100000
D = 1024
B_TOTAL = 4 * 4096

NC = 2
NS = 16
NW = NC * NS

NSPLIT = 4
B_SPLIT = B_TOTAL // NSPLIT
B_PER_W = B_SPLIT // NW
CHUNK = 32
NCHUNK = B_PER_W // CHUNK

BM = 2048


def _sc_gather(table, ids):
    mesh = plsc.VectorSubcoreMesh(core_axis_name="c", subcore_axis_name="s")

    @functools.partial(
        pl.kernel,
        out_type=jax.ShapeDtypeStruct((B_SPLIT, D), jnp.float32),
        mesh=mesh,
        scratch_types=[
            pltpu.VMEM((B_PER_W,), jnp.int32),
            pltpu.VMEM((CHUNK, D), jnp.float32),
            pltpu.VMEM((CHUNK, D), jnp.float32),
            pltpu.SemaphoreType.DMA,
            pltpu.SemaphoreType.DMA,
            pltpu.SemaphoreType.DMA,
            pltpu.SemaphoreType.DMA,
        ],
    )
    def gather_kernel(table_hbm, ids_hbm, out_hbm, idx_v, rows_a, rows_b,
                      gsem_a, gsem_b, wsem_a, wsem_b):
        wid = lax.axis_index("s") * NC + lax.axis_index("c")
        base = wid * B_PER_W
        pltpu.sync_copy(ids_hbm.at[pl.ds(base, B_PER_W)], idx_v)
        bufs = (rows_a, rows_b)
        gsems = (gsem_a, gsem_b)
        wsems = (wsem_a, wsem_b)
        gathers = [None] * NCHUNK
        writes = [None] * NCHUNK
        gathers[0] = pltpu.async_copy(
            table_hbm.at[idx_v.at[pl.ds(0, CHUNK)]], bufs[0], gsems[0]
        )
        for c in range(NCHUNK):
            gathers[c].wait()
            writes[c] = pltpu.async_copy(
                bufs[c % 2], out_hbm.at[pl.ds(base + c * CHUNK, CHUNK)],
                wsems[c % 2]
            )
            if c + 1 < NCHUNK:
                if c >= 1:
                    writes[c - 1].wait()
                gathers[c + 1] = pltpu.async_copy(
                    table_hbm.at[idx_v.at[pl.ds((c + 1) * CHUNK, CHUNK)]],
                    bufs[(c + 1) % 2], gsems[(c + 1) % 2]
                )
        if NCHUNK >= 2:
            writes[NCHUNK - 2].wait()
        writes[NCHUNK - 1].wait()

    return gather_kernel(table, ids)


def _mm_first_body(x_ref, r_ref, o_ref):
    o_ref[...] = lax.dot_general(
        x_ref[...], r_ref[...], (((1,), (0,)), ((), ())),
        preferred_element_type=jnp.float32, precision=lax.Precision.DEFAULT,
    )


def _mm_alias_body(buf_ref, x_ref, r_ref, o_ref):
    del buf_ref
    o_ref[...] = lax.dot_general(
        x_ref[...], r_ref[...], (((1,), (0,)), ((), ())),
        preferred_element_type=jnp.float32, precision=lax.Precision.DEFAULT,
    )


def _tc_matmul_into(buf, x, r, piece):
    nb = B_SPLIT // BM
    off = piece * nb
    out_spec = pl.BlockSpec((BM, D), lambda j: (j + jnp.int32(off), jnp.int32(0)))
    x_spec = pl.BlockSpec((BM, D), lambda j: (j, jnp.int32(0)))
    r_spec = pl.BlockSpec((D, D), lambda j: (jnp.int32(0), jnp.int32(0)))
    out_shape = jax.ShapeDtypeStruct((B_TOTAL, D), jnp.float32)
    if piece == 0:
        return pl.pallas_call(
            _mm_first_body,
            grid=(nb,),
            in_specs=[x_spec, r_spec],
            out_specs=out_spec,
            out_shape=out_shape,
        )(x, r)
    return pl.pallas_call(
        _mm_alias_body,
        grid=(nb,),
        in_specs=[pl.BlockSpec(memory_space=pl.ANY), x_spec, r_spec],
        out_specs=out_spec,
        out_shape=out_shape,
        input_output_aliases={0: 0},
    )(buf, x, r)


def kernel(inp_ids, table, R):
    batch, seq = inp_ids.shape
    ids = inp_ids.reshape(-1).astype(jnp.int32)
    r32 = R.astype(jnp.float32)
    gathered = [
        _sc_gather(table, lax.slice(ids, (k * B_SPLIT,), ((k + 1) * B_SPLIT,)))
        for k in range(NSPLIT)
    ]
    buf = None
    for k in range(NSPLIT):
        buf = _tc_matmul_into(buf, gathered[k], r32, k)
    return buf.reshape(batch, seq, D).astype(table.dtype)


# speedup vs baseline: 1.1900x; 1.1900x over previous
import functools

import jax
import jax.numpy as jnp
from jax import lax
from jax.experimental import pallas as pl
from jax.experimental.pallas import tpu as pltpu
from jax.experimental.pallas import tpu_sc as plsc

VOCAB = 100000
D = 1024
B_TOTAL = 4 * 4096

NC = 2
NS = 16
NW = NC * NS
B_PER_W = B_TOTAL // NW
CHUNK = 32
NCHUNK = B_PER_W // CHUNK
NBUF = 3

BM = 2048


def _sc_gather(table, ids):
    mesh = plsc.VectorSubcoreMesh(core_axis_name="c", subcore_axis_name="s")

    @functools.partial(
        pl.kernel,
        out_type=jax.ShapeDtypeStruct((B_TOTAL, D), jnp.float32),
        mesh=mesh,
        scratch_types=(
            [pltpu.VMEM((B_PER_W,), jnp.int32)]
            + [pltpu.VMEM((CHUNK, D), jnp.float32) for _ in range(NBUF)]
            + [pltpu.SemaphoreType.DMA for _ in range(2 * NBUF)]
        ),
    )
    def gather_kernel(table_hbm, ids_hbm, out_hbm, idx_v, *bufs_sems):
        bufs = bufs_sems[:NBUF]
        gsems = bufs_sems[NBUF:2 * NBUF]
        wsems = bufs_sems[2 * NBUF:]
        wid = lax.axis_index("s") * NC + lax.axis_index("c")
        base = wid * B_PER_W
        pltpu.sync_copy(ids_hbm.at[pl.ds(base, B_PER_W)], idx_v)
        gathers = [None] * NCHUNK
        writes = [None] * NCHUNK
        for c in range(NBUF - 1):
            gathers[c] = pltpu.async_copy(
                table_hbm.at[idx_v.at[pl.ds(c * CHUNK, CHUNK)]],
                bufs[c % NBUF], gsems[c % NBUF]
            )
        for c in range(NCHUNK):
            n = c + NBUF - 1
            if n < NCHUNK:
                if n >= NBUF:
                    writes[n - NBUF].wait()
                gathers[n] = pltpu.async_copy(
                    table_hbm.at[idx_v.at[pl.ds(n * CHUNK, CHUNK)]],
                    bufs[n % NBUF], gsems[n % NBUF]
                )
            gathers[c].wait()
            writes[c] = pltpu.async_copy(
                bufs[c % NBUF], out_hbm.at[pl.ds(base + c * CHUNK, CHUNK)],
                wsems[c % NBUF]
            )
        for c in range(max(0, NCHUNK - NBUF), NCHUNK):
            writes[c].wait()

    return gather_kernel(table, ids)


def _matmul_body(x_ref, r_ref, o_ref):
    o_ref[...] = lax.dot_general(
        x_ref[...], r_ref[...], (((1,), (0,)), ((), ())),
        preferred_element_type=jnp.float32, precision=lax.Precision.DEFAULT,
    )


def _tc_matmul(x, r):
    return pl.pallas_call(
        _matmul_body,
        grid=(B_TOTAL // BM,),
        in_specs=[
            pl.BlockSpec((BM, D), lambda i: (i, jnp.int32(0))),
            pl.BlockSpec((D, D), lambda i: (jnp.int32(0), jnp.int32(0))),
        ],
        out_specs=pl.BlockSpec((BM, D), lambda i: (i, jnp.int32(0))),
        out_shape=jax.ShapeDtypeStruct((B_TOTAL, D), jnp.float32),
    )(x, r)


def kernel(inp_ids, table, R):
    batch, seq = inp_ids.shape
    r32 = R.astype(jnp.float32)
    ids = inp_ids.reshape(-1).astype(jnp.int32)
    gathered = _sc_gather(table, ids)
    out = _tc_matmul(gathered, r32)
    return out.reshape(batch, seq, D).astype(table.dtype)
